# speedup vs baseline: 1.3504x; 1.3504x over previous
import functools

import jax
import jax.numpy as jnp
from jax import lax
from jax.experimental import pallas as pl
from jax.experimental.pallas import tpu as pltpu
from jax.experimental.pallas import tpu_sc as plsc

_ROWS = 50000
_D = 128
_CHUNK = 128
_NC = 2
_NS = 16
_NW = _NC * _NS
_NCHUNKS = -(-_ROWS // _CHUNK)
_PER_W = -(-_NCHUNKS // _NW)
_LAST_BASE = _ROWS - _CHUNK


@functools.partial(
    pl.kernel,
    mesh=plsc.VectorSubcoreMesh(core_axis_name="c", subcore_axis_name="s"),
    out_type=jax.ShapeDtypeStruct((_ROWS, _D), jnp.float32),
    scratch_types=[
        pltpu.VMEM((_CHUNK,), jnp.int32),
        pltpu.VMEM((_CHUNK, _D), jnp.float32),
        pltpu.SemaphoreType.DMA,
    ],
)
def _gather_sc(table_hbm, idx_hbm, out_hbm, idx_v, rows_v, sem):
    wid = lax.axis_index("s") * _NC + lax.axis_index("c")
    for j in range(_PER_W):
        c = wid + _NW * j
        base = jnp.minimum(c * _CHUNK, _LAST_BASE)
        pltpu.sync_copy(idx_hbm.at[pl.ds(base, _CHUNK)], idx_v)
        pltpu.async_copy(table_hbm.at[idx_v], rows_v, sem).wait()
        pltpu.sync_copy(rows_v, out_hbm.at[pl.ds(base, _CHUNK)])


def kernel(features, sel_idx_up):
    idx = sel_idx_up.reshape(-1)
    return _gather_sc(features, idx)


# speedup vs baseline: 1.9799x; 1.4661x over previous
import functools

import jax
import jax.numpy as jnp
from jax import lax
from jax.experimental import pallas as pl
from jax.experimental.pallas import tpu as pltpu
from jax.experimental.pallas import tpu_sc as plsc

_ROWS = 50000
_D = 128
_NW = 32
_PW = 1568
_C = 112
_NCH = _PW // _C
_LAST_W_BASE = _ROWS - _PW


@functools.partial(
    pl.kernel,
    mesh=plsc.VectorSubcoreMesh(core_axis_name="c", subcore_axis_name="s"),
    out_type=jax.ShapeDtypeStruct((_ROWS, _D), jnp.float32),
    scratch_types=[
        pltpu.VMEM((_PW,), jnp.int32),
        pltpu.VMEM((_C, _D), jnp.float32),
        pltpu.VMEM((_C, _D), jnp.float32),
        pltpu.SemaphoreType.DMA,
        pltpu.SemaphoreType.DMA,
        pltpu.SemaphoreType.DMA,
        pltpu.SemaphoreType.DMA,
    ],
)
def _gather_sc(table_hbm, idx_hbm, out_hbm, idx_v, buf0, buf1,
               sg0, sg1, ss0, ss1):
    wid = lax.axis_index("s") * 2 + lax.axis_index("c")
    base_w = jnp.minimum(wid * _PW, _LAST_W_BASE)
    pltpu.sync_copy(idx_hbm.at[pl.ds(base_w, _PW)], idx_v)

    bufs = (buf0, buf1)
    sgs = (sg0, sg1)
    sss = (ss0, ss1)
    gcp = [None, None]
    scp = [None, None]

    def start_gather(j, p):
        return pltpu.async_copy(
            table_hbm.at[idx_v.at[pl.ds(j * _C, _C)]], bufs[p], sgs[p])

    def start_store(j, p):
        return pltpu.async_copy(
            bufs[p], out_hbm.at[pl.ds(base_w + j * _C, _C)], sss[p])

    for j in range(_NCH):
        p = j % 2
        if scp[p] is not None:
            scp[p].wait()
        gcp[p] = start_gather(j, p)
        if j > 0:
            q = (j - 1) % 2
            gcp[q].wait()
            scp[q] = start_store(j - 1, q)

    q = (_NCH - 1) % 2
    gcp[q].wait()
    scp[q] = start_store(_NCH - 1, q)
    scp[1 - q].wait()
    scp[q].wait()


def kernel(features, sel_idx_up):
    idx = sel_idx_up.reshape(-1)
    return _gather_sc(features, idx)


# speedup vs baseline: 2.0134x; 1.0169x over previous
import functools

import jax
import jax.numpy as jnp
from jax import lax
from jax.experimental import pallas as pl
from jax.experimental.pallas import tpu as pltpu
from jax.experimental.pallas import tpu_sc as plsc

_ROWS = 50000
_D = 128
_NW = 32
_PW = 1568
_C = 112
_NCH = _PW // _C
_NB = 4
_LAST_W_BASE = _ROWS - _PW


@functools.partial(
    pl.kernel,
    mesh=plsc.VectorSubcoreMesh(core_axis_name="c", subcore_axis_name="s"),
    out_type=jax.ShapeDtypeStruct((_ROWS, _D), jnp.float32),
    scratch_types=[
        pltpu.VMEM((_PW,), jnp.int32),
        pltpu.VMEM((_NB, _C, _D), jnp.float32),
    ] + [pltpu.SemaphoreType.DMA] * (2 * _NB),
)
def _gather_sc(table_hbm, idx_hbm, out_hbm, idx_v, bufs, *sems):
    sgs = sems[:_NB]
    sss = sems[_NB:]
    wid = lax.axis_index("s") * 2 + lax.axis_index("c")
    base_w = jnp.minimum(wid * _PW, _LAST_W_BASE)
    pltpu.sync_copy(idx_hbm.at[pl.ds(base_w, _PW)], idx_v)

    gcp = [None] * _NB
    scp = [None] * _NB

    def start_gather(j, p):
        return pltpu.async_copy(
            table_hbm.at[idx_v.at[pl.ds(j * _C, _C)]], bufs.at[p], sgs[p])

    def start_store(j, p):
        return pltpu.async_copy(
            bufs.at[p], out_hbm.at[pl.ds(base_w + j * _C, _C)], sss[p])

    for j in range(_NCH):
        p = j % _NB
        if scp[p] is not None:
            scp[p].wait()
        gcp[p] = start_gather(j, p)
        if j >= _NB - 1:
            jq = j - (_NB - 1)
            q = jq % _NB
            gcp[q].wait()
            scp[q] = start_store(jq, q)

    for jq in range(max(_NCH - (_NB - 1), 0), _NCH):
        q = jq % _NB
        gcp[q].wait()
        scp[q] = start_store(jq, q)
    for q in range(_NB):
        if scp[q] is not None:
            scp[q].wait()


def kernel(features, sel_idx_up):
    idx = sel_idx_up.reshape(-1)
    return _gather_sc(features, idx)


# speedup vs baseline: 2.0176x; 1.0021x over previous
import functools

import jax
import jax.numpy as jnp
from jax import lax
from jax.experimental import pallas as pl
from jax.experimental.pallas import tpu as pltpu
from jax.experimental.pallas import tpu_sc as plsc

_ROWS = 50000
_D = 128
_NW = 32
_PW = 1568
_C = 112
_NCH = _PW // _C
_NB = 4
_LAST_W_BASE = _ROWS - _PW
_LOOP_LO = _NB
_LOOP_HI = _NCH - ((_NCH - _NB) % _NB)


@functools.partial(
    pl.kernel,
    mesh=plsc.VectorSubcoreMesh(core_axis_name="c", subcore_axis_name="s"),
    out_type=jax.ShapeDtypeStruct((_ROWS, _D), jnp.float32),
    scratch_types=[
        pltpu.VMEM((_PW,), jnp.int32),
        pltpu.VMEM((_NB, _C, _D), jnp.float32),
    ] + [pltpu.SemaphoreType.DMA] * (2 * _NB),
)
def _gather_sc(table_hbm, idx_hbm, out_hbm, idx_v, bufs, *sems):
    sgs = sems[:_NB]
    sss = sems[_NB:]
    wid = lax.axis_index("s") * 2 + lax.axis_index("c")
    base_w = jnp.minimum(wid * _PW, _LAST_W_BASE)
    pltpu.sync_copy(idx_hbm.at[pl.ds(base_w, _PW)], idx_v)

    def start_gather(j, p):
        pltpu.async_copy(
            table_hbm.at[idx_v.at[pl.ds(j * _C, _C)]], bufs.at[p], sgs[p])

    def start_store(j, p):
        pltpu.async_copy(
            bufs.at[p], out_hbm.at[pl.ds(base_w + j * _C, _C)], sss[p])

    def wait_gather(p):
        pltpu.make_async_copy(
            out_hbm.at[pl.ds(base_w, _C)], bufs.at[p], sgs[p]).wait()

    def wait_store(p):
        pltpu.make_async_copy(
            bufs.at[p], out_hbm.at[pl.ds(base_w, _C)], sss[p]).wait()

    for b in range(_NB):
        start_gather(b, b)
    wait_gather(0)
    start_store(0, 0)

    @pl.loop(_LOOP_LO, _LOOP_HI, step=_NB)
    def _body(j0):
        for b in range(_NB):
            j = j0 + b
            p = b
            q = (b + 1) % _NB
            wait_store(p)
            start_gather(j, p)
            wait_gather(q)
            start_store(j - (_NB - 1), q)

    for j in range(_LOOP_HI, _NCH):
        p = j % _NB
        q = (j + 1) % _NB
        wait_store(p)
        start_gather(j, p)
        wait_gather(q)
        start_store(j - (_NB - 1), q)

    for j in range(_NCH - (_NB - 1), _NCH):
        p = j % _NB
        wait_gather(p)
        start_store(j, p)
    for p in range(_NB):
        wait_store(p)


def kernel(features, sel_idx_up):
    idx = sel_idx_up.reshape(-1)
    return _gather_sc(features, idx)


# speedup vs baseline: 2.0689x; 1.0254x over previous
import functools

import jax
import jax.numpy as jnp
from jax import lax
from jax.experimental import pallas as pl
from jax.experimental.pallas import tpu as pltpu
from jax.experimental.pallas import tpu_sc as plsc

_ROWS = 50000
_D = 128
_NW = 32
_PW = 1568
_C = 112
_NCH = _PW // _C
_NB = 7
_LAST_W_BASE = _ROWS - _PW
_LOOP_LO = _NB
_LOOP_HI = _NCH - ((_NCH - _NB) % _NB)


@functools.partial(
    pl.kernel,
    mesh=plsc.VectorSubcoreMesh(core_axis_name="c", subcore_axis_name="s"),
    out_type=jax.ShapeDtypeStruct((_ROWS, _D), jnp.float32),
    scratch_types=[
        pltpu.VMEM((_PW,), jnp.int32),
        pltpu.VMEM((_NB, _C, _D), jnp.float32),
    ] + [pltpu.SemaphoreType.DMA] * (2 * _NB),
)
def _gather_sc(table_hbm, idx_hbm, out_hbm, idx_v, bufs, *sems):
    sgs = sems[:_NB]
    sss = sems[_NB:]
    wid = lax.axis_index("s") * 2 + lax.axis_index("c")
    base_w = jnp.minimum(wid * _PW, _LAST_W_BASE)
    pltpu.sync_copy(idx_hbm.at[pl.ds(base_w, _PW)], idx_v)

    def start_gather(j, p):
        pltpu.async_copy(
            table_hbm.at[idx_v.at[pl.ds(j * _C, _C)]], bufs.at[p], sgs[p])

    def start_store(j, p):
        pltpu.async_copy(
            bufs.at[p], out_hbm.at[pl.ds(base_w + j * _C, _C)], sss[p])

    def wait_gather(p):
        pltpu.make_async_copy(
            out_hbm.at[pl.ds(base_w, _C)], bufs.at[p], sgs[p]).wait()

    def wait_store(p):
        pltpu.make_async_copy(
            bufs.at[p], out_hbm.at[pl.ds(base_w, _C)], sss[p]).wait()

    for b in range(_NB):
        start_gather(b, b)
    wait_gather(0)
    start_store(0, 0)

    @pl.loop(_LOOP_LO, _LOOP_HI, step=_NB)
    def _body(j0):
        for b in range(_NB):
            j = j0 + b
            p = b
            q = (b + 1) % _NB
            wait_store(p)
            start_gather(j, p)
            wait_gather(q)
            start_store(j - (_NB - 1), q)

    for j in range(_LOOP_HI, _NCH):
        p = j % _NB
        q = (j + 1) % _NB
        wait_store(p)
        start_gather(j, p)
        wait_gather(q)
        start_store(j - (_NB - 1), q)

    for j in range(_NCH - (_NB - 1), _NCH):
        p = j % _NB
        wait_gather(p)
        start_store(j, p)
    for p in range(_NB):
        wait_store(p)


def kernel(features, sel_idx_up):
    idx = sel_idx_up.reshape(-1)
    return _gather_sc(features, idx)


# speedup vs baseline: 2.0714x; 1.0012x over previous
import functools

import jax
import jax.numpy as jnp
from jax import lax
from jax.experimental import pallas as pl
from jax.experimental.pallas import tpu as pltpu
from jax.experimental.pallas import tpu_sc as plsc

_ROWS = 50000
_D = 128
_NW = 32
_PW = 1568
_C = 112
_NCH = _PW // _C
_NB = 8
_LAST_W_BASE = _ROWS - _PW
_LOOP_LO = _NB
_LOOP_HI = _NCH - ((_NCH - _NB) % _NB)


@functools.partial(
    pl.kernel,
    mesh=plsc.VectorSubcoreMesh(core_axis_name="c", subcore_axis_name="s"),
    out_type=jax.ShapeDtypeStruct((_ROWS, _D), jnp.float32),
    scratch_types=[
        pltpu.VMEM((_PW,), jnp.int32),
        pltpu.VMEM((_NB, _C, _D), jnp.float32),
    ] + [pltpu.SemaphoreType.DMA] * (2 * _NB),
)
def _gather_sc(table_hbm, idx_hbm, out_hbm, idx_v, bufs, *sems):
    sgs = sems[:_NB]
    sss = sems[_NB:]
    wid = lax.axis_index("s") * 2 + lax.axis_index("c")
    base_w = jnp.minimum(wid * _PW, _LAST_W_BASE)
    pltpu.sync_copy(idx_hbm.at[pl.ds(base_w, _PW)], idx_v)

    def start_gather(j, p):
        pltpu.async_copy(
            table_hbm.at[idx_v.at[pl.ds(j * _C, _C)]], bufs.at[p], sgs[p])

    def start_store(j, p):
        pltpu.async_copy(
            bufs.at[p], out_hbm.at[pl.ds(base_w + j * _C, _C)], sss[p])

    def wait_gather(p):
        pltpu.make_async_copy(
            out_hbm.at[pl.ds(base_w, _C)], bufs.at[p], sgs[p]).wait()

    def wait_store(p):
        pltpu.make_async_copy(
            bufs.at[p], out_hbm.at[pl.ds(base_w, _C)], sss[p]).wait()

    for b in range(_NB):
        start_gather(b, b)
    wait_gather(0)
    start_store(0, 0)

    @pl.loop(_LOOP_LO, _LOOP_HI, step=_NB)
    def _body(j0):
        for b in range(_NB):
            j = j0 + b
            p = b
            q = (b + 1) % _NB
            wait_store(p)
            start_gather(j, p)
            wait_gather(q)
            start_store(j - (_NB - 1), q)

    for j in range(_LOOP_HI, _NCH):
        p = j % _NB
        q = (j + 1) % _NB
        wait_store(p)
        start_gather(j, p)
        wait_gather(q)
        start_store(j - (_NB - 1), q)

    for j in range(_NCH - (_NB - 1), _NCH):
        p = j % _NB
        wait_gather(p)
        start_store(j, p)
    for p in range(_NB):
        wait_store(p)


def kernel(features, sel_idx_up):
    idx = sel_idx_up.reshape(-1)
    return _gather_sc(features, idx)


# speedup vs baseline: 2.0868x; 1.0075x over previous
import functools

import jax
import jax.numpy as jnp
from jax import lax
from jax.experimental import pallas as pl
from jax.experimental.pallas import tpu as pltpu
from jax.experimental.pallas import tpu_sc as plsc

_ROWS = 50000
_D = 128
_NW = 32
_PW = 1568
_C = 56
_NCH = _PW // _C
_NB = 8
_LAST_W_BASE = _ROWS - _PW
_LOOP_LO = _NB
_LOOP_HI = _NCH - ((_NCH - _NB) % _NB)


@functools.partial(
    pl.kernel,
    mesh=plsc.VectorSubcoreMesh(core_axis_name="c", subcore_axis_name="s"),
    out_type=jax.ShapeDtypeStruct((_ROWS, _D), jnp.float32),
    scratch_types=[
        pltpu.VMEM((_PW,), jnp.int32),
        pltpu.VMEM((_NB, _C, _D), jnp.float32),
    ] + [pltpu.SemaphoreType.DMA] * (2 * _NB),
)
def _gather_sc(table_hbm, idx_hbm, out_hbm, idx_v, bufs, *sems):
    sgs = sems[:_NB]
    sss = sems[_NB:]
    wid = lax.axis_index("s") * 2 + lax.axis_index("c")
    base_w = jnp.minimum(wid * _PW, _LAST_W_BASE)
    pltpu.sync_copy(idx_hbm.at[pl.ds(base_w, _PW)], idx_v)

    def start_gather(j, p):
        pltpu.async_copy(
            table_hbm.at[idx_v.at[pl.ds(j * _C, _C)]], bufs.at[p], sgs[p])

    def start_store(j, p):
        pltpu.async_copy(
            bufs.at[p], out_hbm.at[pl.ds(base_w + j * _C, _C)], sss[p])

    def wait_gather(p):
        pltpu.make_async_copy(
            out_hbm.at[pl.ds(base_w, _C)], bufs.at[p], sgs[p]).wait()

    def wait_store(p):
        pltpu.make_async_copy(
            bufs.at[p], out_hbm.at[pl.ds(base_w, _C)], sss[p]).wait()

    for b in range(_NB):
        start_gather(b, b)
    wait_gather(0)
    start_store(0, 0)

    @pl.loop(_LOOP_LO, _LOOP_HI, step=_NB)
    def _body(j0):
        for b in range(_NB):
            j = j0 + b
            p = b
            q = (b + 1) % _NB
            wait_store(p)
            start_gather(j, p)
            wait_gather(q)
            start_store(j - (_NB - 1), q)

    for j in range(_LOOP_HI, _NCH):
        p = j % _NB
        q = (j + 1) % _NB
        wait_store(p)
        start_gather(j, p)
        wait_gather(q)
        start_store(j - (_NB - 1), q)

    for j in range(_NCH - (_NB - 1), _NCH):
        p = j % _NB
        wait_gather(p)
        start_store(j, p)
    for p in range(_NB):
        wait_store(p)


def kernel(features, sel_idx_up):
    idx = sel_idx_up.reshape(-1)
    return _gather_sc(features, idx)
